# baseline (speedup 1.0000x reference)
import functools

import jax
import jax.numpy as jnp
from jax import lax
from jax.experimental import pallas as pl
from jax.experimental.pallas import tpu as pltpu
from jax.experimental.pallas import tpu_sc as plsc

VOCAB = 1_000_000
HIDDEN = 64
OUT = 128
HALF = OUT // 2
B = 16384
L = 50
NIDX = B * L

_info = plsc.get_sparse_core_info()
NC = _info.num_cores
NS = _info.num_subcores
NW = NC * NS
IDXW = 128
K = 2
CHUNK = K * IDXW
PER_W = NIDX // NW
STEPS = PER_W // CHUNK


def _tc_decode_half(tT, Wh, bh2d):
    NB = 4096

    def body(t_ref, w_ref, b_ref, o_ref):
        o_ref[...] = (
            lax.dot_general(
                t_ref[...], w_ref[...],
                (((0,), (0,)), ((), ())),
                preferred_element_type=jnp.float32,
            )
            + b_ref[...]
        )

    return pl.pallas_call(
        body,
        grid=(pl.cdiv(VOCAB, NB),),
        in_specs=[
            pl.BlockSpec((HIDDEN, NB), lambda i: (0, i)),
            pl.BlockSpec((HIDDEN, HALF), lambda i: (0, 0)),
            pl.BlockSpec((1, HALF), lambda i: (0, 0)),
        ],
        out_specs=pl.BlockSpec((NB, HALF), lambda i: (i, 0)),
        out_shape=jax.ShapeDtypeStruct((VOCAB, HALF), jnp.float32),
        compiler_params=pltpu.CompilerParams(
            dimension_semantics=("arbitrary",),
        ),
    )(tT, Wh, bh2d)


def _make_sc_gather_half(col0):
    mesh = plsc.VectorSubcoreMesh(core_axis_name="c", subcore_axis_name="s")

    @functools.partial(
        pl.kernel,
        mesh=mesh,
        out_type=(),
        scratch_types=[
            pltpu.VMEM((2, K, IDXW), jnp.int32),
            pltpu.VMEM((2, CHUNK, HALF), jnp.float32),
            pltpu.SemaphoreType.DMA,
            pltpu.SemaphoreType.DMA,
            pltpu.SemaphoreType.DMA,
        ],
        compiler_params=pltpu.CompilerParams(use_tc_tiling_on_sc=False),
    )
    def k(t2_hbm, idx_hbm, out_hbm, idx_v, rows_v, gsem, wsem0, wsem1):
        wsems = (wsem0, wsem1)
        wid = lax.axis_index("s") * NC + lax.axis_index("c")
        row0 = wid * (PER_W // IDXW)
        base = wid * PER_W

        def fire_gather(i, b):
            pltpu.sync_copy(idx_hbm.at[pl.ds(row0 + i * K, K)], idx_v.at[b])
            for j in range(K):
                pltpu.async_copy(
                    t2_hbm.at[idx_v.at[b].at[j]],
                    rows_v.at[b].at[pl.ds(j * IDXW, IDXW)],
                    gsem,
                )

        def wait_gather(b):
            for j in range(K):
                pltpu.make_async_copy(
                    t2_hbm.at[idx_v.at[b].at[j]],
                    rows_v.at[b].at[pl.ds(j * IDXW, IDXW)],
                    gsem,
                ).wait()

        def out_slice(i):
            return out_hbm.at[
                pl.ds(base + i * CHUNK, CHUNK), pl.ds(col0, HALF)
            ]

        def start_write(i, b):
            pltpu.async_copy(rows_v.at[b], out_slice(i), wsems[b])

        def wait_write(i, b):
            pltpu.make_async_copy(rows_v.at[b], out_slice(i), wsems[b]).wait()

        fire_gather(0, 0)
        wait_gather(0)
        start_write(0, 0)
        fire_gather(1, 1)

        def pair(i2, carry):
            i_a = 1 + 2 * i2
            wait_gather(1)
            start_write(i_a, 1)
            wait_write(i_a - 1, 0)
            fire_gather(i_a + 1, 0)
            i_b = i_a + 1
            wait_gather(0)
            start_write(i_b, 0)
            wait_write(i_b - 1, 1)
            fire_gather(i_b + 1, 1)
            return carry

        lax.fori_loop(0, (STEPS - 2) // 2, pair, 0)

        wait_gather(1)
        start_write(STEPS - 1, 1)
        wait_write(STEPS - 2, 0)
        wait_write(STEPS - 1, 1)

    return k


_gather_lo = _make_sc_gather_half(0)
_gather_hi = _make_sc_gather_half(HALF)


def kernel(message, table, W, b):
    tT = jnp.transpose(table)
    idxT = jnp.transpose(message).reshape(NIDX // IDXW, IDXW)
    t2a = _tc_decode_half(tT, W[:, :HALF], b[:HALF].reshape(1, HALF))
    t2b = _tc_decode_half(tT, W[:, HALF:], b[HALF:].reshape(1, HALF))
    out_ref = jax.new_ref(pl.empty((NIDX, OUT), jnp.float32))
    _gather_lo(t2a, idxT, out_ref)
    _gather_hi(t2b, idxT, out_ref)
    outT = out_ref[...]
    out = jnp.transpose(outT.reshape(L, B, OUT), (1, 0, 2))
    return out


# speedup vs baseline: 1.3659x; 1.3659x over previous
import functools

import jax
import jax.numpy as jnp
from jax import lax
from jax.experimental import pallas as pl
from jax.experimental.pallas import tpu as pltpu
from jax.experimental.pallas import tpu_sc as plsc

VOCAB = 1_000_000
HIDDEN = 64
OUT = 128
HALF = OUT // 2
B = 16384
L = 50
NIDX = B * L

_info = plsc.get_sparse_core_info()
NC = _info.num_cores
NS = _info.num_subcores
NW = NC * NS
IDXW = 128
K = 2
CHUNK = K * IDXW
PER_W = NIDX // NW
STEPS = PER_W // CHUNK


def _tc_decode_packed(tT, W, b2d):
    NB = 4096

    def body(t_ref, w_ref, b_ref, o_ref):
        full = (
            lax.dot_general(
                t_ref[...], w_ref[...],
                (((0,), (0,)), ((), ())),
                preferred_element_type=jnp.float32,
            )
            + b_ref[...]
        )
        lo = full[:, :HALF]
        hi = full[:, HALF:]
        lou = jax.lax.bitcast_convert_type(lo, jnp.uint32)
        hiu = jax.lax.bitcast_convert_type(hi, jnp.uint32)
        lou = (lou + jnp.uint32(0x8000)) >> jnp.uint32(16)
        hiu = (hiu + jnp.uint32(0x8000)) & jnp.uint32(0xFFFF0000)
        o_ref[...] = jax.lax.bitcast_convert_type(lou | hiu, jnp.int32)

    return pl.pallas_call(
        body,
        grid=(pl.cdiv(VOCAB, NB),),
        in_specs=[
            pl.BlockSpec((HIDDEN, NB), lambda i: (0, i)),
            pl.BlockSpec((HIDDEN, OUT), lambda i: (0, 0)),
            pl.BlockSpec((1, OUT), lambda i: (0, 0)),
        ],
        out_specs=pl.BlockSpec((NB, HALF), lambda i: (i, 0)),
        out_shape=jax.ShapeDtypeStruct((VOCAB, HALF), jnp.int32),
        compiler_params=pltpu.CompilerParams(
            dimension_semantics=("arbitrary",),
        ),
    )(tT, W, b2d)


def _sc_gather_unpack(t2p, idx2d):
    mesh = plsc.VectorSubcoreMesh(core_axis_name="c", subcore_axis_name="s")

    @functools.partial(
        pl.kernel,
        mesh=mesh,
        out_type=jax.ShapeDtypeStruct((NIDX, OUT), jnp.float32),
        scratch_types=[
            pltpu.VMEM((2, K, IDXW), jnp.int32),
            pltpu.VMEM((2, CHUNK, HALF), jnp.int32),
            pltpu.VMEM((2, CHUNK, OUT), jnp.float32),
            pltpu.SemaphoreType.DMA,
            pltpu.SemaphoreType.DMA,
            pltpu.SemaphoreType.DMA,
        ],
        compiler_params=pltpu.CompilerParams(
            use_tc_tiling_on_sc=False, needs_layout_passes=False
        ),
    )
    def k(t2_hbm, idx_hbm, out_hbm, idx_v, rows_p, rows_f, gsem, wsem0, wsem1):
        wsems = (wsem0, wsem1)
        wid = lax.axis_index("s") * NC + lax.axis_index("c")
        row0 = wid * (PER_W // IDXW)
        base = wid * PER_W

        def fire_gather(i, b):
            pltpu.sync_copy(idx_hbm.at[pl.ds(row0 + i * K, K)], idx_v.at[b])
            for j in range(K):
                pltpu.async_copy(
                    t2_hbm.at[idx_v.at[b].at[j]],
                    rows_p.at[b].at[pl.ds(j * IDXW, IDXW)],
                    gsem,
                )

        def wait_gather(b):
            for j in range(K):
                pltpu.make_async_copy(
                    t2_hbm.at[idx_v.at[b].at[j]],
                    rows_p.at[b].at[pl.ds(j * IDXW, IDXW)],
                    gsem,
                ).wait()

        def unpack(b):
            src = rows_p.at[b]
            dst = rows_f.at[b]

            def row(i, carry):
                for g in range(HALF // 16):
                    w = src[i, pl.ds(g * 16, 16)]
                    lo = plsc.bitcast(w << 16, jnp.float32)
                    hi = plsc.bitcast(
                        w & jnp.int32(-65536), jnp.float32
                    )
                    dst[i, pl.ds(g * 16, 16)] = lo
                    dst[i, pl.ds(HALF + g * 16, 16)] = hi
                return carry

            lax.fori_loop(0, CHUNK, row, 0)

        def start_write(i, b):
            pltpu.async_copy(
                rows_f.at[b], out_hbm.at[pl.ds(base + i * CHUNK, CHUNK)],
                wsems[b],
            )

        def wait_write(i, b):
            pltpu.make_async_copy(
                rows_f.at[b], out_hbm.at[pl.ds(base + i * CHUNK, CHUNK)],
                wsems[b],
            ).wait()

        fire_gather(0, 0)
        fire_gather(1, 1)
        wait_gather(0)
        unpack(0)
        start_write(0, 0)
        fire_gather(2, 0)
        wait_gather(1)
        unpack(1)
        start_write(1, 1)
        fire_gather(3, 1)

        def pair(i2, carry):
            i_a = 2 * i2
            wait_gather(0)
            wait_write(i_a - 2, 0)
            unpack(0)
            start_write(i_a, 0)
            fire_gather(i_a + 2, 0)
            i_b = i_a + 1
            wait_gather(1)
            wait_write(i_b - 2, 1)
            unpack(1)
            start_write(i_b, 1)
            fire_gather(i_b + 2, 1)
            return carry

        lax.fori_loop(1, STEPS // 2 - 1, pair, 0)

        wait_gather(0)
        wait_write(STEPS - 4, 0)
        unpack(0)
        start_write(STEPS - 2, 0)
        wait_gather(1)
        wait_write(STEPS - 3, 1)
        unpack(1)
        start_write(STEPS - 1, 1)
        wait_write(STEPS - 2, 0)
        wait_write(STEPS - 1, 1)

    return k(t2p, idx2d)


def kernel(message, table, W, b):
    tT = jnp.transpose(table)
    idxT = jnp.transpose(message).reshape(NIDX // IDXW, IDXW)
    t2p = _tc_decode_packed(tT, W, b.reshape(1, OUT))
    outT = _sc_gather_unpack(t2p, idxT)
    out = jnp.transpose(outT.reshape(L, B, OUT), (1, 0, 2))
    return out


# speedup vs baseline: 2.8639x; 2.0967x over previous
import functools

import jax
import jax.numpy as jnp
from jax import lax
from jax.experimental import pallas as pl
from jax.experimental.pallas import tpu as pltpu
from jax.experimental.pallas import tpu_sc as plsc

VOCAB = 1_000_000
HIDDEN = 64
OUT = 128
B = 16384
L = 50
NIDX = B * L

_info = plsc.get_sparse_core_info()
NC = _info.num_cores
NS = _info.num_subcores
NW = NC * NS
IDXW = 128
K = 2
CHUNK = K * IDXW
PER_W = NIDX // NW
STEPS = PER_W // CHUNK


def _tc_decode_table(tT, W, bias2d):
    NB = 8192

    def body(t_ref, w_ref, b_ref, o_ref):
        o_ref[...] = (
            lax.dot_general(
                t_ref[...], w_ref[...],
                (((0,), (0,)), ((), ())),
                preferred_element_type=jnp.float32,
            )
            + b_ref[...]
        )

    return pl.pallas_call(
        body,
        grid=(pl.cdiv(VOCAB, NB),),
        in_specs=[
            pl.BlockSpec((HIDDEN, NB), lambda i: (0, i)),
            pl.BlockSpec((HIDDEN, OUT), lambda i: (0, 0)),
            pl.BlockSpec((1, OUT), lambda i: (0, 0)),
        ],
        out_specs=pl.BlockSpec((NB, OUT), lambda i: (i, 0)),
        out_shape=jax.ShapeDtypeStruct((VOCAB, OUT), jnp.float32),
        compiler_params=pltpu.CompilerParams(
            dimension_semantics=("arbitrary",),
        ),
    )(tT, W, bias2d)


def _sc_gather(t2, idx2d):
    mesh = plsc.VectorSubcoreMesh(core_axis_name="c", subcore_axis_name="s")

    @functools.partial(
        pl.kernel,
        mesh=mesh,
        out_type=jax.ShapeDtypeStruct((NIDX, OUT), jnp.float32),
        scratch_types=[
            pltpu.VMEM((2, K, IDXW), jnp.int32),
            pltpu.VMEM((2, CHUNK, OUT), jnp.float32),
            pltpu.SemaphoreType.DMA,
            pltpu.SemaphoreType.DMA,
            pltpu.SemaphoreType.DMA,
        ],
        compiler_params=pltpu.CompilerParams(use_tc_tiling_on_sc=False),
    )
    def k(t2_hbm, idx_hbm, out_hbm, idx_v, rows_v, gsem, wsem0, wsem1):
        wsems = (wsem0, wsem1)
        wid = lax.axis_index("s") * NC + lax.axis_index("c")
        row0 = wid * (PER_W // IDXW)
        base = wid * PER_W

        def fire_gather(i, b):
            pltpu.sync_copy(idx_hbm.at[pl.ds(row0 + i * K, K)], idx_v.at[b])
            for j in range(K):
                pltpu.async_copy(
                    t2_hbm.at[idx_v.at[b].at[j]],
                    rows_v.at[b].at[pl.ds(j * IDXW, IDXW)],
                    gsem,
                )

        def wait_gather(b):
            for j in range(K):
                pltpu.make_async_copy(
                    t2_hbm.at[idx_v.at[b].at[j]],
                    rows_v.at[b].at[pl.ds(j * IDXW, IDXW)],
                    gsem,
                ).wait()

        def start_write(i, b):
            pltpu.async_copy(
                rows_v.at[b], out_hbm.at[pl.ds(base + i * CHUNK, CHUNK)], wsems[b]
            )

        def wait_write(i, b):
            pltpu.make_async_copy(
                rows_v.at[b], out_hbm.at[pl.ds(base + i * CHUNK, CHUNK)], wsems[b]
            ).wait()

        fire_gather(0, 0)
        wait_gather(0)
        start_write(0, 0)
        fire_gather(1, 1)

        def pair(i2, carry):
            i_a = 1 + 2 * i2
            wait_gather(1)
            start_write(i_a, 1)
            wait_write(i_a - 1, 0)
            fire_gather(i_a + 1, 0)
            i_b = i_a + 1
            wait_gather(0)
            start_write(i_b, 0)
            wait_write(i_b - 1, 1)
            fire_gather(i_b + 1, 1)
            return carry

        lax.fori_loop(0, (STEPS - 2) // 2, pair, 0)

        wait_gather(1)
        start_write(STEPS - 1, 1)
        wait_write(STEPS - 2, 0)
        wait_write(STEPS - 1, 1)

    return k(t2, idx2d)


def kernel(message, table, W, b):
    tT = jnp.transpose(table)
    idxT = jnp.transpose(message).reshape(NIDX // IDXW, IDXW)
    t2 = _tc_decode_table(tT, W, b.reshape(1, OUT))
    outT = _sc_gather(t2, idxT)
    out = jnp.transpose(outT.reshape(L, B, OUT), (1, 0, 2))
    return out


# speedup vs baseline: 2.9878x; 1.0433x over previous
import functools

import jax
import jax.numpy as jnp
from jax import lax
from jax.experimental import pallas as pl
from jax.experimental.pallas import tpu as pltpu
from jax.experimental.pallas import tpu_sc as plsc

VOCAB = 1_000_000
HIDDEN = 64
OUT = 128
B = 16384
L = 50
NIDX = B * L

_info = plsc.get_sparse_core_info()
NC = _info.num_cores
NS = _info.num_subcores
NW = NC * NS
IDXW = 128
K = 2
CHUNK = K * IDXW
PER_W = NIDX // NW
STEPS = PER_W // CHUNK


def _tc_decode_table(tT, W, bias2d):
    NB = 16384

    def body(t_ref, w_ref, b_ref, o_ref):
        o_ref[...] = (
            lax.dot_general(
                t_ref[...], w_ref[...],
                (((0,), (0,)), ((), ())),
                preferred_element_type=jnp.float32,
            )
            + b_ref[...]
        )

    return pl.pallas_call(
        body,
        grid=(pl.cdiv(VOCAB, NB),),
        in_specs=[
            pl.BlockSpec((HIDDEN, NB), lambda i: (0, i)),
            pl.BlockSpec((HIDDEN, OUT), lambda i: (0, 0)),
            pl.BlockSpec((1, OUT), lambda i: (0, 0)),
        ],
        out_specs=pl.BlockSpec((NB, OUT), lambda i: (i, 0)),
        out_shape=jax.ShapeDtypeStruct((VOCAB, OUT), jnp.float32),
        compiler_params=pltpu.CompilerParams(
            dimension_semantics=("arbitrary",),
        ),
    )(tT, W, bias2d)


def _sc_gather(t2, idx2d):
    mesh = plsc.VectorSubcoreMesh(core_axis_name="c", subcore_axis_name="s")

    @functools.partial(
        pl.kernel,
        mesh=mesh,
        out_type=jax.ShapeDtypeStruct((NIDX, OUT), jnp.float32),
        scratch_types=[
            pltpu.VMEM((2, K, IDXW), jnp.int32),
            pltpu.VMEM((2, CHUNK, OUT), jnp.float32),
            pltpu.SemaphoreType.DMA,
            pltpu.SemaphoreType.DMA,
            pltpu.SemaphoreType.DMA,
        ],
        compiler_params=pltpu.CompilerParams(use_tc_tiling_on_sc=False),
    )
    def k(t2_hbm, idx_hbm, out_hbm, idx_v, rows_v, gsem, wsem0, wsem1):
        wsems = (wsem0, wsem1)
        wid = lax.axis_index("s") * NC + lax.axis_index("c")
        row0 = wid * (PER_W // IDXW)
        base = wid * PER_W

        def fire_gather(i, b):
            pltpu.sync_copy(idx_hbm.at[pl.ds(row0 + i * K, K)], idx_v.at[b])
            for j in range(K):
                pltpu.async_copy(
                    t2_hbm.at[idx_v.at[b].at[j]],
                    rows_v.at[b].at[pl.ds(j * IDXW, IDXW)],
                    gsem,
                )

        def wait_gather(b):
            for j in range(K):
                pltpu.make_async_copy(
                    t2_hbm.at[idx_v.at[b].at[j]],
                    rows_v.at[b].at[pl.ds(j * IDXW, IDXW)],
                    gsem,
                ).wait()

        def start_write(i, b):
            pltpu.async_copy(
                rows_v.at[b], out_hbm.at[pl.ds(base + i * CHUNK, CHUNK)], wsems[b]
            )

        def wait_write(i, b):
            pltpu.make_async_copy(
                rows_v.at[b], out_hbm.at[pl.ds(base + i * CHUNK, CHUNK)], wsems[b]
            ).wait()

        fire_gather(0, 0)
        wait_gather(0)
        start_write(0, 0)
        fire_gather(1, 1)

        def pair(i2, carry):
            i_a = 1 + 2 * i2
            wait_gather(1)
            start_write(i_a, 1)
            wait_write(i_a - 1, 0)
            fire_gather(i_a + 1, 0)
            i_b = i_a + 1
            wait_gather(0)
            start_write(i_b, 0)
            wait_write(i_b - 1, 1)
            fire_gather(i_b + 1, 1)
            return carry

        lax.fori_loop(0, (STEPS - 2) // 2, pair, 0)

        wait_gather(1)
        start_write(STEPS - 1, 1)
        wait_write(STEPS - 2, 0)
        wait_write(STEPS - 1, 1)

    return k(t2, idx2d)


def kernel(message, table, W, b):
    tT = jnp.transpose(table)
    idxT = jnp.transpose(message).reshape(NIDX // IDXW, IDXW)
    t2 = _tc_decode_table(tT, W, b.reshape(1, OUT))
    outT = _sc_gather(t2, idxT)
    out = jnp.transpose(outT.reshape(L, B, OUT), (1, 0, 2))
    return out


# speedup vs baseline: 3.0223x; 1.0116x over previous
import functools

import jax
import jax.numpy as jnp
from jax import lax
from jax.experimental import pallas as pl
from jax.experimental.pallas import tpu as pltpu
from jax.experimental.pallas import tpu_sc as plsc

VOCAB = 1_000_000
HIDDEN = 64
OUT = 128
B = 16384
L = 50
NIDX = B * L

_info = plsc.get_sparse_core_info()
NC = _info.num_cores
NS = _info.num_subcores
NW = NC * NS
IDXW = 128
K = 2
CHUNK = K * IDXW
PER_W = NIDX // NW
STEPS = PER_W // CHUNK


def _tc_decode_table(tT, W, bias2d):
    NB = 32768

    def body(t_ref, w_ref, b_ref, o_ref):
        o_ref[...] = (
            lax.dot_general(
                t_ref[...], w_ref[...],
                (((0,), (0,)), ((), ())),
                preferred_element_type=jnp.float32,
            )
            + b_ref[...]
        )

    return pl.pallas_call(
        body,
        grid=(pl.cdiv(VOCAB, NB),),
        in_specs=[
            pl.BlockSpec((HIDDEN, NB), lambda i: (0, i)),
            pl.BlockSpec((HIDDEN, OUT), lambda i: (0, 0)),
            pl.BlockSpec((1, OUT), lambda i: (0, 0)),
        ],
        out_specs=pl.BlockSpec((NB, OUT), lambda i: (i, 0)),
        out_shape=jax.ShapeDtypeStruct((VOCAB, OUT), jnp.float32),
        compiler_params=pltpu.CompilerParams(
            dimension_semantics=("arbitrary",),
        ),
    )(tT, W, bias2d)


def _sc_gather(t2, idx2d):
    mesh = plsc.VectorSubcoreMesh(core_axis_name="c", subcore_axis_name="s")

    @functools.partial(
        pl.kernel,
        mesh=mesh,
        out_type=jax.ShapeDtypeStruct((NIDX, OUT), jnp.float32),
        scratch_types=[
            pltpu.VMEM((2, K, IDXW), jnp.int32),
            pltpu.VMEM((2, CHUNK, OUT), jnp.float32),
            pltpu.SemaphoreType.DMA,
            pltpu.SemaphoreType.DMA,
            pltpu.SemaphoreType.DMA,
        ],
        compiler_params=pltpu.CompilerParams(use_tc_tiling_on_sc=False),
    )
    def k(t2_hbm, idx_hbm, out_hbm, idx_v, rows_v, gsem, wsem0, wsem1):
        wsems = (wsem0, wsem1)
        wid = lax.axis_index("s") * NC + lax.axis_index("c")
        row0 = wid * (PER_W // IDXW)
        base = wid * PER_W

        def fire_gather(i, b):
            pltpu.sync_copy(idx_hbm.at[pl.ds(row0 + i * K, K)], idx_v.at[b])
            for j in range(K):
                pltpu.async_copy(
                    t2_hbm.at[idx_v.at[b].at[j]],
                    rows_v.at[b].at[pl.ds(j * IDXW, IDXW)],
                    gsem,
                )

        def wait_gather(b):
            for j in range(K):
                pltpu.make_async_copy(
                    t2_hbm.at[idx_v.at[b].at[j]],
                    rows_v.at[b].at[pl.ds(j * IDXW, IDXW)],
                    gsem,
                ).wait()

        def start_write(i, b):
            pltpu.async_copy(
                rows_v.at[b], out_hbm.at[pl.ds(base + i * CHUNK, CHUNK)], wsems[b]
            )

        def wait_write(i, b):
            pltpu.make_async_copy(
                rows_v.at[b], out_hbm.at[pl.ds(base + i * CHUNK, CHUNK)], wsems[b]
            ).wait()

        fire_gather(0, 0)
        wait_gather(0)
        start_write(0, 0)
        fire_gather(1, 1)

        def pair(i2, carry):
            i_a = 1 + 2 * i2
            wait_gather(1)
            start_write(i_a, 1)
            wait_write(i_a - 1, 0)
            fire_gather(i_a + 1, 0)
            i_b = i_a + 1
            wait_gather(0)
            start_write(i_b, 0)
            wait_write(i_b - 1, 1)
            fire_gather(i_b + 1, 1)
            return carry

        lax.fori_loop(0, (STEPS - 2) // 2, pair, 0)

        wait_gather(1)
        start_write(STEPS - 1, 1)
        wait_write(STEPS - 2, 0)
        wait_write(STEPS - 1, 1)

    return k(t2, idx2d)


def kernel(message, table, W, b):
    tT = jnp.transpose(table)
    idxT = jnp.transpose(message).reshape(NIDX // IDXW, IDXW)
    t2 = _tc_decode_table(tT, W, b.reshape(1, OUT))
    outT = _sc_gather(t2, idxT)
    out = jnp.transpose(outT.reshape(L, B, OUT), (1, 0, 2))
    return out


# speedup vs baseline: 3.2006x; 1.0590x over previous
import functools

import jax
import jax.numpy as jnp
from jax import lax
from jax.experimental import pallas as pl
from jax.experimental.pallas import tpu as pltpu
from jax.experimental.pallas import tpu_sc as plsc

VOCAB = 1_000_000
HIDDEN = 64
OUT = 128
B = 16384
L = 50
NIDX = B * L

_info = plsc.get_sparse_core_info()
NC = _info.num_cores
NS = _info.num_subcores
NW = NC * NS
IDXW = 128
K = 2
CHUNK = K * IDXW
PER_W = NIDX // NW
STEPS = PER_W // CHUNK


def _tc_decode_table(tT, W, bias2d):
    NB = 32768

    def body(t_ref, w_ref, b_ref, o_ref):
        o_ref[...] = (
            lax.dot_general(
                t_ref[...], w_ref[...],
                (((0,), (0,)), ((), ())),
                preferred_element_type=jnp.float32,
            )
            + b_ref[...]
        )

    return pl.pallas_call(
        body,
        grid=(pl.cdiv(VOCAB, NB),),
        in_specs=[
            pl.BlockSpec((HIDDEN, NB), lambda i: (0, i)),
            pl.BlockSpec((HIDDEN, OUT), lambda i: (0, 0)),
            pl.BlockSpec((1, OUT), lambda i: (0, 0)),
        ],
        out_specs=pl.BlockSpec((NB, OUT), lambda i: (i, 0)),
        out_shape=jax.ShapeDtypeStruct((VOCAB, OUT), jnp.float32),
        compiler_params=pltpu.CompilerParams(
            dimension_semantics=("arbitrary",),
        ),
    )(tT, W, bias2d)


def _sc_gather(t2, idx2d):
    mesh = plsc.VectorSubcoreMesh(core_axis_name="c", subcore_axis_name="s")

    @functools.partial(
        pl.kernel,
        mesh=mesh,
        out_type=jax.ShapeDtypeStruct((NIDX, OUT), jnp.float32),
        scratch_types=[
            pltpu.VMEM((PER_W // IDXW, IDXW), jnp.int32),
            pltpu.VMEM((2, CHUNK, OUT), jnp.float32),
            pltpu.SemaphoreType.DMA,
            pltpu.SemaphoreType.DMA,
            pltpu.SemaphoreType.DMA,
        ],
        compiler_params=pltpu.CompilerParams(use_tc_tiling_on_sc=False),
    )
    def k(t2_hbm, idx_hbm, out_hbm, idx_v, rows_v, gsem, wsem0, wsem1):
        wsems = (wsem0, wsem1)
        wid = lax.axis_index("s") * NC + lax.axis_index("c")
        row0 = wid * (PER_W // IDXW)
        base = wid * PER_W

        pltpu.sync_copy(idx_hbm.at[pl.ds(row0, PER_W // IDXW)], idx_v)

        def fire_gather(i, b):
            for j in range(K):
                pltpu.async_copy(
                    t2_hbm.at[idx_v.at[i * K + j]],
                    rows_v.at[b].at[pl.ds(j * IDXW, IDXW)],
                    gsem,
                )

        def wait_gather(i, b):
            for j in range(K):
                pltpu.make_async_copy(
                    t2_hbm.at[idx_v.at[i * K + j]],
                    rows_v.at[b].at[pl.ds(j * IDXW, IDXW)],
                    gsem,
                ).wait()

        def start_write(i, b):
            pltpu.async_copy(
                rows_v.at[b], out_hbm.at[pl.ds(base + i * CHUNK, CHUNK)], wsems[b]
            )

        def wait_write(i, b):
            pltpu.make_async_copy(
                rows_v.at[b], out_hbm.at[pl.ds(base + i * CHUNK, CHUNK)], wsems[b]
            ).wait()

        fire_gather(0, 0)
        wait_gather(0, 0)
        start_write(0, 0)
        fire_gather(1, 1)

        def pair(i2, carry):
            i_a = 1 + 2 * i2
            wait_gather(i_a, 1)
            start_write(i_a, 1)
            wait_write(i_a - 1, 0)
            fire_gather(i_a + 1, 0)
            i_b = i_a + 1
            wait_gather(i_b, 0)
            start_write(i_b, 0)
            wait_write(i_b - 1, 1)
            fire_gather(i_b + 1, 1)
            return carry

        lax.fori_loop(0, (STEPS - 2) // 2, pair, 0)

        wait_gather(STEPS - 1, 1)
        start_write(STEPS - 1, 1)
        wait_write(STEPS - 2, 0)
        wait_write(STEPS - 1, 1)

    return k(t2, idx2d)


def kernel(message, table, W, b):
    tT = jnp.transpose(table)
    idxT = jnp.transpose(message).reshape(NIDX // IDXW, IDXW)
    t2 = _tc_decode_table(tT, W, b.reshape(1, OUT))
    outT = _sc_gather(t2, idxT)
    out = jnp.transpose(outT.reshape(L, B, OUT), (1, 0, 2))
    return out
